# baseline (speedup 1.0000x reference)
import functools

import jax
import jax.numpy as jnp
from jax.experimental import pallas as pl

_EPS = 1e-10
_N_TOK = 128
_VOCAB = 100000
_LANE = 128
_ROWS = 8
_NSTEP = _N_TOK // _ROWS
_NFULL = _VOCAB // _LANE
_TAIL = _VOCAB - _NFULL * _LANE
_BIG = 2**30


def _body(logits_ref, invt_ref, gum_ref, out_ref):
    invt = invt_ref[...]
    bases = [g * _LANE for g in range(_NFULL)]
    if _TAIL:
        bases.append(_VOCAB - _LANE)

    m = None
    a = None
    for base in bases:
        blk = logits_ref[:, base:base + _LANE] * invt + gum_ref[:, base:base + _LANE]
        if m is None:
            m = blk
            a = jnp.full((_ROWS, _LANE), 0, jnp.int32)
        else:
            upd = blk > m
            m = jnp.where(upd, blk, m)
            a = jnp.where(upd, jnp.int32(base), a)

    best = m.max(axis=1, keepdims=True)
    lane = jax.lax.broadcasted_iota(jnp.int32, m.shape, 1)
    cand = a + lane
    out_ref[...] = jnp.min(
        jnp.where(m == best, cand, _BIG), axis=1, keepdims=True
    )


@functools.partial(jax.jit, static_argnames=())
def kernel(logits, temperatures, exponential):
    invt = (1.0 / jnp.clip(temperatures, _EPS, None)).reshape(_N_TOK, 1)
    gum = -jnp.log(exponential)
    out = pl.pallas_call(
        _body,
        grid=(_NSTEP,),
        in_specs=[
            pl.BlockSpec((_ROWS, _VOCAB), lambda j: (j, 0)),
            pl.BlockSpec((_ROWS, 1), lambda j: (j, 0)),
            pl.BlockSpec((1, _VOCAB), lambda j: (0, 0)),
        ],
        out_specs=pl.BlockSpec((_ROWS, 1), lambda j: (j, 0)),
        out_shape=jax.ShapeDtypeStruct((_N_TOK, 1), jnp.int32),
    )(logits, invt, gum)
    return out.reshape(_N_TOK)


# speedup vs baseline: 1.0523x; 1.0523x over previous
import functools

import jax
import jax.numpy as jnp
from jax.experimental import pallas as pl

_EPS = 1e-10
_N_TOK = 128
_VOCAB = 100000
_LANE = 128
_ROWS = 8
_NSLICE = 4
_STEP_ROWS = _ROWS * _NSLICE
_NSTEP = _N_TOK // _STEP_ROWS
_NFULL = _VOCAB // _LANE
_BIG = 2**30


def _scan_rows(logits_blk, invt_blk, gum_ref):
    bases = list(range(0, _NFULL * _LANE, _LANE))
    if _VOCAB % _LANE:
        bases.append(_VOCAB - _LANE)

    m = None
    a = None
    for base in bases:
        blk = logits_blk[:, base:base + _LANE] * invt_blk + gum_ref[:, base:base + _LANE]
        if m is None:
            m = blk
            a = jnp.zeros((_ROWS, _LANE), jnp.int32)
        else:
            upd = blk > m
            m = jnp.where(upd, blk, m)
            a = jnp.where(upd, jnp.int32(base), a)

    best = m.max(axis=1, keepdims=True)
    lane = jax.lax.broadcasted_iota(jnp.int32, m.shape, 1)
    cand = a + lane
    return jnp.min(jnp.where(m == best, cand, _BIG), axis=1, keepdims=True)


def _body(*refs):
    logit_refs = refs[:_NSLICE]
    invt_ref, gum_ref, out_ref = refs[_NSLICE:]
    for k in range(_NSLICE):
        out_ref[k * _ROWS:(k + 1) * _ROWS, :] = _scan_rows(
            logit_refs[k][...],
            invt_ref[k * _ROWS:(k + 1) * _ROWS, :],
            gum_ref,
        )


@functools.partial(jax.jit, static_argnames=())
def kernel(logits, temperatures, exponential):
    invt = (1.0 / jnp.clip(temperatures, _EPS, None)).reshape(_N_TOK, 1)
    gum = -jnp.log(exponential)

    def _lmap(k):
        return lambda j: (_NSLICE * j + k, 0)

    out = pl.pallas_call(
        _body,
        grid=(_NSTEP,),
        in_specs=[
            pl.BlockSpec((_ROWS, _VOCAB), _lmap(k)) for k in range(_NSLICE)
        ] + [
            pl.BlockSpec((_STEP_ROWS, 1), lambda j: (j, 0)),
            pl.BlockSpec((1, _VOCAB), lambda j: (0, 0)),
        ],
        out_specs=pl.BlockSpec((_STEP_ROWS, 1), lambda j: (j, 0)),
        out_shape=jax.ShapeDtypeStruct((_N_TOK, 1), jnp.int32),
    )(*([logits] * _NSLICE), invt, gum)
    return out.reshape(_N_TOK)


# speedup vs baseline: 2.5056x; 2.3810x over previous
import functools

import jax
import jax.numpy as jnp
from jax.experimental import pallas as pl
from jax.experimental.pallas import tpu as pltpu

_EPS = 1e-10
_N_TOK = 128
_VOCAB = 100000
_CHUNK = 32
_BLK_V = 20000
_NSTEP = _VOCAB // _BLK_V
_NCOL = _VOCAB // _CHUNK
_COLS_STEP = _BLK_V // _CHUNK
_BIG = 2**30


def _body(at_ref, invt_ref, gum_ref, out_ref, accv_ref, acci_ref):
    j = pl.program_id(0)

    @pl.when(j == 0)
    def _():
        accv_ref[...] = jnp.full((_CHUNK, _N_TOK), -jnp.inf, jnp.float32)
        acci_ref[...] = jnp.zeros((_CHUNK, _N_TOK), jnp.int32)

    invt = invt_ref[...]
    m = accv_ref[...]
    a = acci_ref[...]
    base0 = j * _BLK_V
    for q in range(_COLS_STEP):
        s = at_ref[q * _CHUNK:(q + 1) * _CHUNK, :] * invt + gum_ref[0, :, q:q + 1]
        upd = s > m
        m = jnp.where(upd, s, m)
        a = jnp.where(upd, base0 + q * _CHUNK, a)
    accv_ref[...] = m
    acci_ref[...] = a

    @pl.when(j == _NSTEP - 1)
    def _():
        best = m.max(axis=0, keepdims=True)
        sub = jax.lax.broadcasted_iota(jnp.int32, m.shape, 0)
        cand = a + sub
        out_ref[...] = jnp.min(
            jnp.where(m == best, cand, _BIG), axis=0, keepdims=True
        )


@functools.partial(jax.jit, static_argnames=())
def kernel(logits, temperatures, exponential):
    at = logits.T
    invt = (1.0 / jnp.clip(temperatures, _EPS, None)).reshape(1, _N_TOK)
    gum = (-jnp.log(exponential)).reshape(_NSTEP, _COLS_STEP, _CHUNK).transpose(0, 2, 1)
    out = pl.pallas_call(
        _body,
        grid=(_NSTEP,),
        in_specs=[
            pl.BlockSpec((_BLK_V, _N_TOK), lambda j: (j, 0)),
            pl.BlockSpec((1, _N_TOK), lambda j: (0, 0)),
            pl.BlockSpec((1, _CHUNK, _COLS_STEP), lambda j: (j, 0, 0)),
        ],
        out_specs=pl.BlockSpec((1, _N_TOK), lambda j: (0, 0)),
        out_shape=jax.ShapeDtypeStruct((1, _N_TOK), jnp.int32),
        scratch_shapes=[
            pltpu.VMEM((_CHUNK, _N_TOK), jnp.float32),
            pltpu.VMEM((_CHUNK, _N_TOK), jnp.int32),
        ],
    )(at, invt, gum)
    return out.reshape(_N_TOK)
